# baseline (speedup 1.0000x reference)
import functools

import jax
import jax.numpy as jnp
from jax import lax
from jax.experimental import pallas as pl
from jax.experimental.pallas import tpu as pltpu
from jax.experimental.pallas import tpu_sc as plsc

NC = 2
NS = 16
LANES = 16
SP = 208
HALFP = SP // 2

_PERM = (
    [2 * i for i in range(16)]
    + [2 * i + 1 for i in range(16)]
    + [32 + 2 * i for i in range(16)]
    + [32 + 2 * i + 1 for i in range(16)]
)


def _split_bf16(x32):
    return plsc.unpack(x32, format=plsc.PackFormat.INTERLEAVED,
                       preferred_element_type=jnp.float32)


def _sc_avg(tokens_flat, table_bf):
    V, D = table_bf.shape
    B = tokens_flat.shape[0] // SP
    NW = NC * NS
    BPW = B // NW

    mesh = plsc.VectorSubcoreMesh(core_axis_name="c", subcore_axis_name="s")

    @functools.partial(
        pl.kernel,
        out_type=jax.ShapeDtypeStruct((B, D), jnp.float32),
        mesh=mesh,
        scratch_types=[
            pltpu.VMEM((BPW * SP,), jnp.int32),
            pltpu.VMEM((2, SP, D), jnp.bfloat16),
            pltpu.VMEM((1, D), jnp.bfloat16),
            pltpu.VMEM((BPW, D), jnp.float32),
            pltpu.SemaphoreType.DMA((2,)),
        ],
        compiler_params=pltpu.CompilerParams(
            use_tc_tiling_on_sc=False, needs_layout_passes=False),
    )
    def sc_kernel(tok_hbm, table_hbm, out_hbm, idx_v, rows_v, row0_v, out_v,
                  sems):
        wid = lax.axis_index("s") * NC + lax.axis_index("c")
        base = wid * BPW
        pltpu.sync_copy(tok_hbm.at[pl.ds(base * SP, BPW * SP)], idx_v)
        pltpu.sync_copy(table_hbm.at[pl.ds(0, 1)], row0_v)

        r0 = []
        for k in range(D // 32):
            r0.extend(_split_bf16(row0_v[0, pl.ds(32 * k, 32)]))

        def issue(r, buf):
            for j in range(2):
                pltpu.async_copy(
                    table_hbm.at[idx_v.at[pl.ds(r * SP + j * HALFP, HALFP)]],
                    rows_v.at[buf, pl.ds(j * HALFP, HALFP)],
                    sems.at[buf],
                )

        def drain(buf):
            for j in range(2):
                pltpu.make_async_copy(
                    table_hbm.at[idx_v.at[pl.ds(j * HALFP, HALFP)]],
                    rows_v.at[buf, pl.ds(j * HALFP, HALFP)],
                    sems.at[buf],
                ).wait()

        issue(0, 0)

        def pair_body(i, carry):
            for b in range(2):
                r = 2 * i + b

                @pl.when(r + 1 < BPW)
                def _():
                    issue(r + 1, 1 - b)

                def cnt_body(k, nz):
                    t16 = idx_v[pl.ds(r * SP + 16 * k, 16)]
                    return nz + plsc.all_reduce_population_count(t16 == 0)

                nz = lax.fori_loop(0, SP // 16, cnt_body,
                                   jnp.zeros((LANES,), jnp.int32))

                drain(b)

                def tok_body(t, accs):
                    out = []
                    for k in range(D // 32):
                        e, o = _split_bf16(rows_v[b, t, pl.ds(32 * k, 32)])
                        out.append(accs[2 * k] + e)
                        out.append(accs[2 * k + 1] + o)
                    return tuple(out)

                accs = lax.fori_loop(
                    0, SP, tok_body,
                    tuple(jnp.zeros((LANES,), jnp.float32)
                          for _ in range(D // LANES)),
                )

                nzf = nz.astype(jnp.float32)
                inv = 1.0 / (jnp.float32(SP) - nzf)
                for k, (acc, r0v) in enumerate(zip(accs, r0)):
                    out_v[r, pl.ds(LANES * k, LANES)] = (acc - nzf * r0v) * inv
            return carry

        lax.fori_loop(0, BPW // 2, pair_body, 0)
        pltpu.sync_copy(out_v, out_hbm.at[pl.ds(base, BPW)])

    return sc_kernel(tokens_flat, table_bf)


def _tc_mlp(avg, W1, b1, W2, b2, W3, b3, W4, b4):
    B, D = avg.shape
    H = W1.shape[1]
    O = W2.shape[1]
    BLK = 1024

    def body(avg_ref, W1r, b1r, W2r, b2r, W3r, b3r, W4r, b4r,
             loc_ref, scale_ref):
        a = avg_ref[...]
        h1 = jnp.maximum(
            jnp.dot(a, W1r[...], preferred_element_type=jnp.float32)
            + b1r[...], 0.0)
        loc_ref[...] = (
            jnp.dot(h1, W2r[...], preferred_element_type=jnp.float32)
            + b2r[...])
        h2 = jnp.maximum(
            jnp.dot(a, W3r[...], preferred_element_type=jnp.float32)
            + b3r[...], 0.0)
        scale_ref[...] = jnp.exp(
            jnp.dot(h2, W4r[...], preferred_element_type=jnp.float32)
            + b4r[...])

    full = lambda shape: pl.BlockSpec(shape, lambda i: (0, 0))
    return pl.pallas_call(
        body,
        grid=(B // BLK,),
        in_specs=[
            pl.BlockSpec((BLK, D), lambda i: (i, 0)),
            full((D, H)), full((1, H)),
            full((H, O)), full((1, O)),
            full((D, H)), full((1, H)),
            full((H, O)), full((1, O)),
        ],
        out_specs=[
            pl.BlockSpec((BLK, O), lambda i: (i, 0)),
            pl.BlockSpec((BLK, O), lambda i: (i, 0)),
        ],
        out_shape=[
            jax.ShapeDtypeStruct((B, O), jnp.float32),
            jax.ShapeDtypeStruct((B, O), jnp.float32),
        ],
    )(avg, W1, b1, W2, b2, W3, b3, W4, b4)


def kernel(tokens, table, W1, b1, W2, b2, W3, b3, W4, b4):
    B, S = tokens.shape
    tokens_p = jnp.concatenate(
        [tokens, jnp.zeros((B, SP - S), jnp.int32)], axis=1).reshape(-1)
    table_bf = table.astype(jnp.bfloat16)
    avg = _sc_avg(tokens_p, table_bf)
    loc, scale = _tc_mlp(
        avg, W1[_PERM, :], b1.reshape(1, -1), W2, b2.reshape(1, -1),
        W3[_PERM, :], b3.reshape(1, -1), W4, b4.reshape(1, -1))
    return (loc, scale)
